# baseline (speedup 1.0000x reference)
import jax
import jax.numpy as jnp
from jax import lax
from jax.experimental import pallas as pl
from jax.experimental.pallas import tpu as pltpu

B = 256
CW_DIM = 2048
ED = 64
K = 1024
DC = CW_DIM // ED
RING = 4


def _vq_all(cwq_ref, cb_ref, x2_ref, c2_ref, oh_hbm, cw_ref, scr, sems):
    iota = lax.broadcasted_iota(jnp.int32, (B, K), 1)
    handles = []
    for d in range(DC):
        x = cwq_ref[:, d * ED:(d + 1) * ED]
        cb = cb_ref[d]
        c2 = c2_ref[d]
        xc = lax.dot_general(x, cb, (((1,), (1,)), ((), ())),
                             preferred_element_type=jnp.float32)
        dist = x2_ref[:, d:d + 1] - 2.0 * xc + c2
        m = jnp.min(dist, axis=1, keepdims=True)
        idx = jnp.min(jnp.where(dist == m, iota, K), axis=1, keepdims=True)
        oh = (iota == idx).astype(jnp.float32)
        r = d % RING
        if d >= RING:
            handles[d - RING].wait()
        scr[r, :, 0, :] = oh
        h = pltpu.make_async_copy(scr.at[r], oh_hbm.at[:, pl.ds(d, 1), :],
                                  sems.at[r])
        h.start()
        handles.append(h)
        cwe = lax.dot_general(oh, cb, (((1,), (0,)), ((), ())),
                              preferred_element_type=jnp.float32)
        cw_ref[:, d * ED:(d + 1) * ED] = x + (cwe - x)
    for h in handles[-RING:]:
        h.wait()


def kernel(cw_q, codebook):
    x = cw_q.reshape(B, DC, ED)
    x2 = jnp.sum(x * x, axis=-1)
    c2 = jnp.sum(codebook * codebook, axis=-1)[:, None, :]

    one_hot, cw = pl.pallas_call(
        _vq_all,
        grid=(1,),
        in_specs=[
            pl.BlockSpec((B, CW_DIM), lambda i: (0, 0)),
            pl.BlockSpec((DC, K, ED), lambda i: (0, 0, 0)),
            pl.BlockSpec((B, DC), lambda i: (0, 0)),
            pl.BlockSpec((DC, 1, K), lambda i: (0, 0, 0)),
        ],
        out_specs=[
            pl.BlockSpec(memory_space=pl.ANY),
            pl.BlockSpec((B, CW_DIM), lambda i: (0, 0)),
        ],
        out_shape=[
            jax.ShapeDtypeStruct((B, DC, K), jnp.float32),
            jax.ShapeDtypeStruct((B, CW_DIM), jnp.float32),
        ],
        scratch_shapes=[
            pltpu.VMEM((RING, B, 1, K), jnp.float32),
            pltpu.SemaphoreType.DMA((RING,)),
        ],
    )(cw_q, codebook, x2, c2)

    return (cw, one_hot)


# speedup vs baseline: 1.1261x; 1.1261x over previous
import jax
import jax.numpy as jnp
from jax import lax
from jax.experimental import pallas as pl
from jax.experimental.pallas import tpu as pltpu

B = 256
CW_DIM = 2048
ED = 64
K = 1024
DC = CW_DIM // ED
DPS = 8
STEPS = DC // DPS


def _vq_step(cwq_ref, cb_ref, x2_ref, c2_ref, oh_ref, cw_ref):
    s = pl.program_id(0)
    row = lax.broadcasted_iota(jnp.int32, (DC, DPS), 0)
    col = lax.broadcasted_iota(jnp.int32, (DC, DPS), 1)
    sel = (row == DPS * s + col).astype(jnp.float32)
    x2blk = lax.dot_general(x2_ref[...], sel, (((1,), (0,)), ((), ())),
                            precision=lax.Precision.HIGHEST,
                            preferred_element_type=jnp.float32)
    iota = lax.broadcasted_iota(jnp.int32, (B, K), 1)
    for j in range(DPS):
        x = cwq_ref[:, j * ED:(j + 1) * ED]
        cb = cb_ref[j]
        c2 = c2_ref[j]
        xc = lax.dot_general(x, cb, (((1,), (1,)), ((), ())),
                             preferred_element_type=jnp.float32)
        dist = x2blk[:, j:j + 1] - 2.0 * xc + c2
        m = jnp.min(dist, axis=1, keepdims=True)
        idx = jnp.min(jnp.where(dist == m, iota, K), axis=1, keepdims=True)
        oh = (iota == idx).astype(jnp.float32)
        oh_ref[:, j, :] = oh
        cwe = lax.dot_general(oh, cb, (((1,), (0,)), ((), ())),
                              preferred_element_type=jnp.float32)
        cw_ref[:, j * ED:(j + 1) * ED] = x + (cwe - x)


def kernel(cw_q, codebook):
    x = cw_q.reshape(B, DC, ED)
    x2 = jnp.sum(x * x, axis=-1)
    c2 = jnp.sum(codebook * codebook, axis=-1)[:, None, :]

    one_hot, cw = pl.pallas_call(
        _vq_step,
        grid=(STEPS,),
        in_specs=[
            pl.BlockSpec((B, DPS * ED), lambda d: (0, d)),
            pl.BlockSpec((DPS, K, ED), lambda d: (d, 0, 0)),
            pl.BlockSpec((B, DC), lambda d: (0, 0)),
            pl.BlockSpec((DPS, 1, K), lambda d: (d, 0, 0)),
        ],
        out_specs=[
            pl.BlockSpec((B, DPS, K), lambda d: (0, d, 0)),
            pl.BlockSpec((B, DPS * ED), lambda d: (0, d)),
        ],
        out_shape=[
            jax.ShapeDtypeStruct((B, DC, K), jnp.float32),
            jax.ShapeDtypeStruct((B, CW_DIM), jnp.float32),
        ],
        compiler_params=pltpu.CompilerParams(
            dimension_semantics=("parallel",),
        ),
    )(cw_q, codebook, x2, c2)

    return (cw, one_hot)
